# baseline (speedup 1.0000x reference)
import functools

import jax
import jax.numpy as jnp
from jax import lax
from jax.experimental import pallas as pl
from jax.experimental.pallas import tpu as pltpu
from jax.experimental.pallas import tpu_sc as plsc

N_FEATURES = 1000000
OUTPUT_DIM = 64
BATCH = 4096
SEQ_LEN = 200

NC = 2
NS = 16
NW = NC * NS

BS = BATCH * SEQ_LEN
ROWS_W = BS // NW
K = 4 * SEQ_LEN
NCH = ROWS_W // K
LANES = 16
D_VREGS = OUTPUT_DIM // LANES


def _make_kernel():
    mesh = plsc.VectorSubcoreMesh(core_axis_name="c", subcore_axis_name="s")

    @functools.partial(
        pl.kernel,
        out_type=jax.ShapeDtypeStruct((BS, OUTPUT_DIM), jnp.float32),
        mesh=mesh,
        scratch_types=[
            pltpu.VMEM((SEQ_LEN, OUTPUT_DIM), jnp.float32),
            pltpu.VMEM((K,), jnp.int32),
            pltpu.VMEM((K, OUTPUT_DIM), jnp.float32),
            pltpu.SemaphoreType.DMA,
        ],
        compiler_params=pltpu.CompilerParams(use_tc_tiling_on_sc=False),
    )
    def k(idx_hbm, table_hbm, pos_hbm, out_hbm, pos_v, idx_v, rows_v, sem):
        wid = lax.axis_index("s") * NC + lax.axis_index("c")
        base = wid * ROWS_W
        pltpu.sync_copy(pos_hbm, pos_v)

        def chunk(g, carry):
            off = base + g * K
            pltpu.sync_copy(idx_hbm.at[pl.ds(off, K)], idx_v)
            pltpu.async_copy(table_hbm.at[idx_v], rows_v, sem).wait()

            def prow(p, c2):
                for j in range(D_VREGS):
                    pv = pos_v[p, pl.ds(j * LANES, LANES)]
                    for t in range(K // SEQ_LEN):
                        r = t * SEQ_LEN + p
                        rows_v[r, pl.ds(j * LANES, LANES)] += pv
                return c2

            lax.fori_loop(0, SEQ_LEN, prow, 0)
            pltpu.sync_copy(rows_v, out_hbm.at[pl.ds(off, K)])
            return carry

        lax.fori_loop(0, NCH, chunk, 0)

    return k


_kernel = _make_kernel()


def kernel(inputs, emb_table, pos_table):
    idx_flat = inputs.reshape(BS)
    out = _kernel(idx_flat, emb_table, pos_table)
    return out.reshape(BATCH, SEQ_LEN, OUTPUT_DIM)
